# baseline (speedup 1.0000x reference)
import functools

import jax
import jax.numpy as jnp
from jax import lax
from jax.experimental import pallas as pl
from jax.experimental.pallas import tpu as pltpu
from jax.experimental.pallas import tpu_sc as plsc

NC = 2
NS = 16
CH = 2048
DEGW = 16


def _sc_mesh():
  return plsc.VectorSubcoreMesh(core_axis_name="c", subcore_axis_name="s")


def _make_deg_kernel(K, n_pad):
  rpt = n_pad // NS

  @functools.partial(
      pl.kernel,
      out_type=jax.ShapeDtypeStruct((NC, n_pad, DEGW), jnp.float32),
      mesh=_sc_mesh(),
      scratch_types=[
          pltpu.VMEM((K, 128), jnp.int32),
          pltpu.VMEM((128, DEGW), jnp.float32),
          pltpu.VMEM_SHARED((n_pad, DEGW), jnp.float32),
      ],
      compiler_params=pltpu.CompilerParams(use_tc_tiling_on_sc=False),
  )
  def deg_kernel(dst_hbm, zeros_hbm, ones_hbm, degp_hbm, dst_v, ones_v, acc_sh):
    c = lax.axis_index("c")
    s = lax.axis_index("s")
    wid = s * NC + c
    pltpu.sync_copy(zeros_hbm.at[pl.ds(rpt * s, rpt)], acc_sh.at[pl.ds(rpt * s, rpt)])
    pltpu.sync_copy(ones_hbm, ones_v)
    pltpu.sync_copy(dst_hbm.at[wid], dst_v)
    plsc.subcore_barrier()

    def step(j, carry):
      pltpu.sync_copy(ones_v, acc_sh.at[dst_v.at[j]], add=True)
      return carry

    lax.fori_loop(0, K, step, 0)
    plsc.subcore_barrier()
    pltpu.sync_copy(acc_sh.at[pl.ds(rpt * s, rpt)],
                    degp_hbm.at[c, pl.ds(rpt * s, rpt)])

  return deg_kernel


def _make_colsplit_kernel(n_chunks, n_pad, d):
  assert d % NS == 0
  cpt = d // NS

  sl = cpt * n_pad

  @functools.partial(
      pl.kernel,
      out_type=jax.ShapeDtypeStruct((NC, d * n_pad), jnp.float32),
      mesh=_sc_mesh(),
      scratch_types=[
          pltpu.VMEM((sl,), jnp.float32),
          pltpu.VMEM((sl,), jnp.float32),
          pltpu.VMEM((2, CH), jnp.int32),
          pltpu.VMEM((2, CH), jnp.int32),
          pltpu.SemaphoreType.DMA((2,)),
          pltpu.SemaphoreType.DMA((2,)),
      ],
      compiler_params=pltpu.CompilerParams(use_tc_tiling_on_sc=False,
                                           needs_layout_passes=False),
  )
  def colsplit_kernel(st_hbm, src_hbm, dst_hbm, zeros_hbm, aggp_hbm,
                      tab_v, acc_v, src_v, dst_v, ssem, dsem):
    c = lax.axis_index("c")
    t = lax.axis_index("s")
    pltpu.async_copy(src_hbm.at[c, 0], src_v.at[0], ssem.at[0])
    pltpu.async_copy(dst_hbm.at[c, 0], dst_v.at[0], dsem.at[0])
    pltpu.sync_copy(st_hbm.at[pl.ds(t * sl, sl)], tab_v)
    pltpu.sync_copy(zeros_hbm.at[pl.ds(t * sl, sl)], acc_v)

    def chunk_step(j, carry):
      p = lax.rem(j, 2)
      q = lax.rem(j + 1, 2)
      pltpu.make_async_copy(src_hbm.at[c, j], src_v.at[p], ssem.at[p]).wait()
      pltpu.make_async_copy(dst_hbm.at[c, j], dst_v.at[p], dsem.at[p]).wait()

      @pl.when(j + 1 < n_chunks)
      def _():
        pltpu.async_copy(src_hbm.at[c, j + 1], src_v.at[q], ssem.at[q])
        pltpu.async_copy(dst_hbm.at[c, j + 1], dst_v.at[q], dsem.at[q])

      @plsc.parallel_loop(0, CH // 16, unroll=8)
      def _body(i):
        sv = src_v[p, pl.ds(i * 16, 16)]
        dv = dst_v[p, pl.ds(i * 16, 16)]
        for col in range(cpt):
          if col == 0:
            si, di = sv, dv
          else:
            off = jnp.full((16,), col * n_pad, jnp.int32)
            si, di = sv + off, dv + off
          vals = plsc.load_gather(tab_v, [si])
          plsc.addupdate_scatter(acc_v, [di], vals)

      return carry

    lax.fori_loop(0, n_chunks, chunk_step, 0)
    pltpu.sync_copy(acc_v, aggp_hbm.at[c, pl.ds(t * sl, sl)])

  return colsplit_kernel


def _dense1_body(degp_ref, x_ref, w1_ref, eye_ref, s1t_ref, s1_ref, dinv_ref):
  deg = degp_ref[0, :, 0:1] + degp_ref[1, :, 0:1] + 1.0
  dinv = lax.rsqrt(deg)
  dinv_ref[...] = dinv
  h1 = jnp.dot(x_ref[...], w1_ref[...], preferred_element_type=jnp.float32)
  s1 = h1 * dinv
  s1_ref[...] = s1
  s1t_ref[...] = lax.dot_general(eye_ref[...], s1, (((1,), (1,)), ((), ())),
                                 preferred_element_type=jnp.float32,
                                 precision=lax.Precision.HIGHEST)


def _mid_body(aggp_ref, s1_ref, dinv_ref, b1_ref, w2_ref, eye_h_ref,
              eye_o_ref, s2t_ref, s2_ref):
  dinv = dinv_ref[...]
  agg_t = aggp_ref[0] + aggp_ref[1]
  agg = lax.dot_general(agg_t, eye_h_ref[...], (((0,), (0,)), ((), ())),
                        preferred_element_type=jnp.float32,
                        precision=lax.Precision.HIGHEST)
  out1 = jnp.maximum(dinv * (agg + s1_ref[...]) + b1_ref[...], 0.0)
  s2 = jnp.dot(out1, w2_ref[...], preferred_element_type=jnp.float32) * dinv
  s2_ref[...] = s2
  s2t_ref[...] = lax.dot_general(eye_o_ref[...], s2, (((1,), (1,)), ((), ())),
                                 preferred_element_type=jnp.float32,
                                 precision=lax.Precision.HIGHEST)


def _final_body(aggp_ref, s2_ref, dinv_ref, b2_ref, eye_ref, z_ref):
  n = z_ref.shape[0]
  agg_t = aggp_ref[0] + aggp_ref[1]
  agg = lax.dot_general(agg_t, eye_ref[...], (((0,), (0,)), ((), ())),
                        preferred_element_type=jnp.float32,
                        precision=lax.Precision.HIGHEST)
  zfull = dinv_ref[...] * (agg + s2_ref[...]) + b2_ref[...]
  z_ref[...] = zfull[:n, :]


def kernel(x, edge_index, W1, b1, W2, b2):
  n, d_in = x.shape
  d_hid = W1.shape[1]
  d_out = W2.shape[1]
  e = edge_index.shape[1]

  n_chunks = -(-e // (NC * CH))
  e_pad = NC * n_chunks * CH
  n_pad = -(-(n + 1) // 128) * 128

  pad = e_pad - e
  src = jnp.concatenate([edge_index[0], jnp.zeros((pad,), jnp.int32)])
  dst = jnp.concatenate([edge_index[1], jnp.full((pad,), n, jnp.int32)])

  L = n_chunks * CH
  src_r = (src.reshape(NC, 16, L // 16).transpose(0, 2, 1)
           .reshape(NC, n_chunks, CH))
  dst_r = (dst.reshape(NC, 16, L // 16).transpose(0, 2, 1)
           .reshape(NC, n_chunks, CH))

  K = -(-e // (NC * NS * 128))
  e_pad_d = NC * NS * K * 128
  dst_d = jnp.concatenate([edge_index[1], jnp.full((e_pad_d - e,), n, jnp.int32)])
  dst_dr = dst_d.reshape(NC * NS, K, 128)

  zeros_deg = jnp.zeros((n_pad, DEGW), jnp.float32)
  ones_col = jnp.ones((128, DEGW), jnp.float32)
  degp = _make_deg_kernel(K, n_pad)(dst_dr, zeros_deg, ones_col)

  x_pad = jnp.pad(x, ((0, n_pad - n), (0, 0)))
  eye_h = jnp.eye(d_hid, dtype=jnp.float32)
  eye_o = jnp.eye(d_out, dtype=jnp.float32)

  s1t, s1, dinv = pl.pallas_call(
      _dense1_body,
      out_shape=[jax.ShapeDtypeStruct((d_hid, n_pad), jnp.float32),
                 jax.ShapeDtypeStruct((n_pad, d_hid), jnp.float32),
                 jax.ShapeDtypeStruct((n_pad, 1), jnp.float32)],
  )(degp, x_pad, W1, eye_h)

  zeros_h = jnp.zeros((d_hid * n_pad,), jnp.float32)
  aggp1 = _make_colsplit_kernel(n_chunks, n_pad, d_hid)(
      s1t.reshape(-1), src_r, dst_r, zeros_h).reshape(NC, d_hid, n_pad)

  s2t, s2 = pl.pallas_call(
      _mid_body,
      out_shape=[jax.ShapeDtypeStruct((d_out, n_pad), jnp.float32),
                 jax.ShapeDtypeStruct((n_pad, d_out), jnp.float32)],
  )(aggp1, s1, dinv, b1.reshape(1, d_hid), W2, eye_h, eye_o)

  zeros_o = jnp.zeros((d_out * n_pad,), jnp.float32)
  aggp2 = _make_colsplit_kernel(n_chunks, n_pad, d_out)(
      s2t.reshape(-1), src_r, dst_r, zeros_o).reshape(NC, d_out, n_pad)

  z = pl.pallas_call(
      _final_body,
      out_shape=jax.ShapeDtypeStruct((n, d_out), jnp.float32),
  )(aggp2, s2, dinv, b2.reshape(1, d_out), eye_o)

  return z


# speedup vs baseline: 1.0814x; 1.0814x over previous
import functools

import jax
import jax.numpy as jnp
from jax import lax
from jax.experimental import pallas as pl
from jax.experimental.pallas import tpu as pltpu
from jax.experimental.pallas import tpu_sc as plsc

NC = 2
NS = 16
CH = 4096
DEGW = 16


def _sc_mesh():
  return plsc.VectorSubcoreMesh(core_axis_name="c", subcore_axis_name="s")


def _make_deg_kernel(K, n_pad):
  rpt = n_pad // NS

  @functools.partial(
      pl.kernel,
      out_type=jax.ShapeDtypeStruct((NC, n_pad, DEGW), jnp.float32),
      mesh=_sc_mesh(),
      scratch_types=[
          pltpu.VMEM((K, 128), jnp.int32),
          pltpu.VMEM((128, DEGW), jnp.float32),
          pltpu.VMEM_SHARED((n_pad, DEGW), jnp.float32),
      ],
      compiler_params=pltpu.CompilerParams(use_tc_tiling_on_sc=False),
  )
  def deg_kernel(dst_hbm, zeros_hbm, ones_hbm, degp_hbm, dst_v, ones_v, acc_sh):
    c = lax.axis_index("c")
    s = lax.axis_index("s")
    wid = s * NC + c
    pltpu.sync_copy(zeros_hbm.at[pl.ds(rpt * s, rpt)], acc_sh.at[pl.ds(rpt * s, rpt)])
    pltpu.sync_copy(ones_hbm, ones_v)
    pltpu.sync_copy(dst_hbm.at[wid], dst_v)
    plsc.subcore_barrier()

    def step(j, carry):
      pltpu.sync_copy(ones_v, acc_sh.at[dst_v.at[j]], add=True)
      return carry

    lax.fori_loop(0, K, step, 0)
    plsc.subcore_barrier()
    pltpu.sync_copy(acc_sh.at[pl.ds(rpt * s, rpt)],
                    degp_hbm.at[c, pl.ds(rpt * s, rpt)])

  return deg_kernel


def _make_colsplit_kernel(n_chunks, n_pad, d):
  assert d % NS == 0
  cpt = d // NS

  sl = cpt * n_pad

  @functools.partial(
      pl.kernel,
      out_type=jax.ShapeDtypeStruct((NC, d * n_pad), jnp.float32),
      mesh=_sc_mesh(),
      scratch_types=[
          pltpu.VMEM((sl,), jnp.float32),
          pltpu.VMEM((sl,), jnp.float32),
          pltpu.VMEM((2, CH), jnp.int32),
          pltpu.VMEM((2, CH), jnp.int32),
          pltpu.SemaphoreType.DMA((2,)),
          pltpu.SemaphoreType.DMA((2,)),
      ],
      compiler_params=pltpu.CompilerParams(use_tc_tiling_on_sc=False,
                                           needs_layout_passes=False),
  )
  def colsplit_kernel(st_hbm, src_hbm, dst_hbm, zeros_hbm, aggp_hbm,
                      tab_v, acc_v, src_v, dst_v, ssem, dsem):
    c = lax.axis_index("c")
    t = lax.axis_index("s")
    pltpu.async_copy(src_hbm.at[c, 0], src_v.at[0], ssem.at[0])
    pltpu.async_copy(dst_hbm.at[c, 0], dst_v.at[0], dsem.at[0])
    pltpu.sync_copy(st_hbm.at[pl.ds(t * sl, sl)], tab_v)
    pltpu.sync_copy(zeros_hbm.at[pl.ds(t * sl, sl)], acc_v)

    def chunk_step(j, carry):
      p = lax.rem(j, 2)
      q = lax.rem(j + 1, 2)
      pltpu.make_async_copy(src_hbm.at[c, j], src_v.at[p], ssem.at[p]).wait()
      pltpu.make_async_copy(dst_hbm.at[c, j], dst_v.at[p], dsem.at[p]).wait()

      @pl.when(j + 1 < n_chunks)
      def _():
        pltpu.async_copy(src_hbm.at[c, j + 1], src_v.at[q], ssem.at[q])
        pltpu.async_copy(dst_hbm.at[c, j + 1], dst_v.at[q], dsem.at[q])

      @plsc.parallel_loop(0, CH // 16, unroll=8)
      def _body(i):
        sv = src_v[p, pl.ds(i * 16, 16)]
        dv = dst_v[p, pl.ds(i * 16, 16)]
        for col in range(cpt):
          if col == 0:
            si, di = sv, dv
          else:
            off = jnp.full((16,), col * n_pad, jnp.int32)
            si, di = sv + off, dv + off
          vals = plsc.load_gather(tab_v, [si])
          plsc.addupdate_scatter(acc_v, [di], vals)

      return carry

    lax.fori_loop(0, n_chunks, chunk_step, 0)
    pltpu.sync_copy(acc_v, aggp_hbm.at[c, pl.ds(t * sl, sl)])

  return colsplit_kernel


def _dense1_body(degp_ref, x_ref, w1_ref, eye_ref, s1t_ref, s1_ref, dinv_ref):
  deg = degp_ref[0, :, 0:1] + degp_ref[1, :, 0:1] + 1.0
  dinv = lax.rsqrt(deg)
  dinv_ref[...] = dinv
  h1 = jnp.dot(x_ref[...], w1_ref[...], preferred_element_type=jnp.float32)
  s1 = h1 * dinv
  s1_ref[...] = s1
  s1t_ref[...] = lax.dot_general(eye_ref[...], s1, (((1,), (1,)), ((), ())),
                                 preferred_element_type=jnp.float32,
                                 precision=lax.Precision.HIGHEST)


def _mid_body(aggp_ref, s1_ref, dinv_ref, b1_ref, w2_ref, eye_h_ref,
              eye_o_ref, s2t_ref, s2_ref):
  dinv = dinv_ref[...]
  agg_t = aggp_ref[0] + aggp_ref[1]
  agg = lax.dot_general(agg_t, eye_h_ref[...], (((0,), (0,)), ((), ())),
                        preferred_element_type=jnp.float32,
                        precision=lax.Precision.HIGHEST)
  out1 = jnp.maximum(dinv * (agg + s1_ref[...]) + b1_ref[...], 0.0)
  s2 = jnp.dot(out1, w2_ref[...], preferred_element_type=jnp.float32) * dinv
  s2_ref[...] = s2
  s2t_ref[...] = lax.dot_general(eye_o_ref[...], s2, (((1,), (1,)), ((), ())),
                                 preferred_element_type=jnp.float32,
                                 precision=lax.Precision.HIGHEST)


def _final_body(aggp_ref, s2_ref, dinv_ref, b2_ref, eye_ref, z_ref):
  n = z_ref.shape[0]
  agg_t = aggp_ref[0] + aggp_ref[1]
  agg = lax.dot_general(agg_t, eye_ref[...], (((0,), (0,)), ((), ())),
                        preferred_element_type=jnp.float32,
                        precision=lax.Precision.HIGHEST)
  zfull = dinv_ref[...] * (agg + s2_ref[...]) + b2_ref[...]
  z_ref[...] = zfull[:n, :]


def kernel(x, edge_index, W1, b1, W2, b2):
  n, d_in = x.shape
  d_hid = W1.shape[1]
  d_out = W2.shape[1]
  e = edge_index.shape[1]

  n_chunks = -(-e // (NC * CH))
  e_pad = NC * n_chunks * CH
  n_pad = -(-(n + 1) // 128) * 128

  pad = e_pad - e
  src = jnp.concatenate([edge_index[0], jnp.zeros((pad,), jnp.int32)])
  dst = jnp.concatenate([edge_index[1], jnp.full((pad,), n, jnp.int32)])
  src_r = src.reshape(NC, n_chunks, CH)
  dst_r = dst.reshape(NC, n_chunks, CH)

  K = -(-e // (NC * NS * 128))
  e_pad_d = NC * NS * K * 128
  dst_d = jnp.concatenate([edge_index[1], jnp.full((e_pad_d - e,), n, jnp.int32)])
  dst_dr = dst_d.reshape(NC * NS, K, 128)

  zeros_deg = jnp.zeros((n_pad, DEGW), jnp.float32)
  ones_col = jnp.ones((128, DEGW), jnp.float32)
  degp = _make_deg_kernel(K, n_pad)(dst_dr, zeros_deg, ones_col)

  x_pad = jnp.pad(x, ((0, n_pad - n), (0, 0)))
  eye_h = jnp.eye(d_hid, dtype=jnp.float32)
  eye_o = jnp.eye(d_out, dtype=jnp.float32)

  s1t, s1, dinv = pl.pallas_call(
      _dense1_body,
      out_shape=[jax.ShapeDtypeStruct((d_hid, n_pad), jnp.float32),
                 jax.ShapeDtypeStruct((n_pad, d_hid), jnp.float32),
                 jax.ShapeDtypeStruct((n_pad, 1), jnp.float32)],
  )(degp, x_pad, W1, eye_h)

  zeros_h = jnp.zeros((d_hid * n_pad,), jnp.float32)
  aggp1 = _make_colsplit_kernel(n_chunks, n_pad, d_hid)(
      s1t.reshape(-1), src_r, dst_r, zeros_h).reshape(NC, d_hid, n_pad)

  s2t, s2 = pl.pallas_call(
      _mid_body,
      out_shape=[jax.ShapeDtypeStruct((d_out, n_pad), jnp.float32),
                 jax.ShapeDtypeStruct((n_pad, d_out), jnp.float32)],
  )(aggp1, s1, dinv, b1.reshape(1, d_hid), W2, eye_h, eye_o)

  zeros_o = jnp.zeros((d_out * n_pad,), jnp.float32)
  aggp2 = _make_colsplit_kernel(n_chunks, n_pad, d_out)(
      s2t.reshape(-1), src_r, dst_r, zeros_o).reshape(NC, d_out, n_pad)

  z = pl.pallas_call(
      _final_body,
      out_shape=jax.ShapeDtypeStruct((n, d_out), jnp.float32),
  )(aggp2, s2, dinv, b2.reshape(1, d_out), eye_o)

  return z


# speedup vs baseline: 1.0931x; 1.0108x over previous
import functools

import jax
import jax.numpy as jnp
from jax import lax
from jax.experimental import pallas as pl
from jax.experimental.pallas import tpu as pltpu
from jax.experimental.pallas import tpu_sc as plsc

NC = 2
NS = 16
CH = 8192
DEGW = 16


def _sc_mesh():
  return plsc.VectorSubcoreMesh(core_axis_name="c", subcore_axis_name="s")


def _make_deg_kernel(K, n_pad):
  rpt = n_pad // NS

  @functools.partial(
      pl.kernel,
      out_type=jax.ShapeDtypeStruct((NC, n_pad, DEGW), jnp.float32),
      mesh=_sc_mesh(),
      scratch_types=[
          pltpu.VMEM((K, 128), jnp.int32),
          pltpu.VMEM((128, DEGW), jnp.float32),
          pltpu.VMEM_SHARED((n_pad, DEGW), jnp.float32),
      ],
      compiler_params=pltpu.CompilerParams(use_tc_tiling_on_sc=False),
  )
  def deg_kernel(dst_hbm, zeros_hbm, ones_hbm, degp_hbm, dst_v, ones_v, acc_sh):
    c = lax.axis_index("c")
    s = lax.axis_index("s")
    wid = s * NC + c
    pltpu.sync_copy(zeros_hbm.at[pl.ds(rpt * s, rpt)], acc_sh.at[pl.ds(rpt * s, rpt)])
    pltpu.sync_copy(ones_hbm, ones_v)
    pltpu.sync_copy(dst_hbm.at[wid], dst_v)
    plsc.subcore_barrier()

    def step(j, carry):
      pltpu.sync_copy(ones_v, acc_sh.at[dst_v.at[j]], add=True)
      return carry

    lax.fori_loop(0, K, step, 0)
    plsc.subcore_barrier()
    pltpu.sync_copy(acc_sh.at[pl.ds(rpt * s, rpt)],
                    degp_hbm.at[c, pl.ds(rpt * s, rpt)])

  return deg_kernel


def _make_colsplit_kernel(n_chunks, n_pad, d):
  assert d % NS == 0
  cpt = d // NS

  sl = cpt * n_pad

  @functools.partial(
      pl.kernel,
      out_type=jax.ShapeDtypeStruct((NC, d * n_pad), jnp.float32),
      mesh=_sc_mesh(),
      scratch_types=[
          pltpu.VMEM((sl,), jnp.float32),
          pltpu.VMEM((sl,), jnp.float32),
          pltpu.VMEM((2, CH), jnp.int32),
          pltpu.VMEM((2, CH), jnp.int32),
          pltpu.SemaphoreType.DMA((2,)),
          pltpu.SemaphoreType.DMA((2,)),
      ],
      compiler_params=pltpu.CompilerParams(use_tc_tiling_on_sc=False,
                                           needs_layout_passes=False),
  )
  def colsplit_kernel(st_hbm, src_hbm, dst_hbm, zeros_hbm, aggp_hbm,
                      tab_v, acc_v, src_v, dst_v, ssem, dsem):
    c = lax.axis_index("c")
    t = lax.axis_index("s")
    pltpu.async_copy(src_hbm.at[c, 0], src_v.at[0], ssem.at[0])
    pltpu.async_copy(dst_hbm.at[c, 0], dst_v.at[0], dsem.at[0])
    pltpu.sync_copy(st_hbm.at[pl.ds(t * sl, sl)], tab_v)
    pltpu.sync_copy(zeros_hbm.at[pl.ds(t * sl, sl)], acc_v)

    def chunk_step(j, carry):
      p = lax.rem(j, 2)
      q = lax.rem(j + 1, 2)
      pltpu.make_async_copy(src_hbm.at[c, j], src_v.at[p], ssem.at[p]).wait()
      pltpu.make_async_copy(dst_hbm.at[c, j], dst_v.at[p], dsem.at[p]).wait()

      @pl.when(j + 1 < n_chunks)
      def _():
        pltpu.async_copy(src_hbm.at[c, j + 1], src_v.at[q], ssem.at[q])
        pltpu.async_copy(dst_hbm.at[c, j + 1], dst_v.at[q], dsem.at[q])

      @plsc.parallel_loop(0, CH // 16, unroll=8)
      def _body(i):
        sv = src_v[p, pl.ds(i * 16, 16)]
        dv = dst_v[p, pl.ds(i * 16, 16)]
        for col in range(cpt):
          if col == 0:
            si, di = sv, dv
          else:
            off = jnp.full((16,), col * n_pad, jnp.int32)
            si, di = sv + off, dv + off
          vals = plsc.load_gather(tab_v, [si])
          plsc.addupdate_scatter(acc_v, [di], vals)

      return carry

    lax.fori_loop(0, n_chunks, chunk_step, 0)
    pltpu.sync_copy(acc_v, aggp_hbm.at[c, pl.ds(t * sl, sl)])

  return colsplit_kernel


def _dense1_body(degp_ref, x_ref, w1_ref, eye_ref, s1t_ref, s1_ref, dinv_ref):
  deg = degp_ref[0, :, 0:1] + degp_ref[1, :, 0:1] + 1.0
  dinv = lax.rsqrt(deg)
  dinv_ref[...] = dinv
  h1 = jnp.dot(x_ref[...], w1_ref[...], preferred_element_type=jnp.float32)
  s1 = h1 * dinv
  s1_ref[...] = s1
  s1t_ref[...] = lax.dot_general(eye_ref[...], s1, (((1,), (1,)), ((), ())),
                                 preferred_element_type=jnp.float32,
                                 precision=lax.Precision.HIGHEST)


def _mid_body(aggp_ref, s1_ref, dinv_ref, b1_ref, w2_ref, eye_h_ref,
              eye_o_ref, s2t_ref, s2_ref):
  dinv = dinv_ref[...]
  agg_t = aggp_ref[0] + aggp_ref[1]
  agg = lax.dot_general(agg_t, eye_h_ref[...], (((0,), (0,)), ((), ())),
                        preferred_element_type=jnp.float32,
                        precision=lax.Precision.HIGHEST)
  out1 = jnp.maximum(dinv * (agg + s1_ref[...]) + b1_ref[...], 0.0)
  s2 = jnp.dot(out1, w2_ref[...], preferred_element_type=jnp.float32) * dinv
  s2_ref[...] = s2
  s2t_ref[...] = lax.dot_general(eye_o_ref[...], s2, (((1,), (1,)), ((), ())),
                                 preferred_element_type=jnp.float32,
                                 precision=lax.Precision.HIGHEST)


def _final_body(aggp_ref, s2_ref, dinv_ref, b2_ref, eye_ref, z_ref):
  n = z_ref.shape[0]
  agg_t = aggp_ref[0] + aggp_ref[1]
  agg = lax.dot_general(agg_t, eye_ref[...], (((0,), (0,)), ((), ())),
                        preferred_element_type=jnp.float32,
                        precision=lax.Precision.HIGHEST)
  zfull = dinv_ref[...] * (agg + s2_ref[...]) + b2_ref[...]
  z_ref[...] = zfull[:n, :]


def kernel(x, edge_index, W1, b1, W2, b2):
  n, d_in = x.shape
  d_hid = W1.shape[1]
  d_out = W2.shape[1]
  e = edge_index.shape[1]

  n_chunks = -(-e // (NC * CH))
  e_pad = NC * n_chunks * CH
  n_pad = -(-(n + 1) // 128) * 128

  pad = e_pad - e
  src = jnp.concatenate([edge_index[0], jnp.zeros((pad,), jnp.int32)])
  dst = jnp.concatenate([edge_index[1], jnp.full((pad,), n, jnp.int32)])
  src_r = src.reshape(NC, n_chunks, CH)
  dst_r = dst.reshape(NC, n_chunks, CH)

  K = -(-e // (NC * NS * 128))
  e_pad_d = NC * NS * K * 128
  dst_d = jnp.concatenate([edge_index[1], jnp.full((e_pad_d - e,), n, jnp.int32)])
  dst_dr = dst_d.reshape(NC * NS, K, 128)

  zeros_deg = jnp.zeros((n_pad, DEGW), jnp.float32)
  ones_col = jnp.ones((128, DEGW), jnp.float32)
  degp = _make_deg_kernel(K, n_pad)(dst_dr, zeros_deg, ones_col)

  x_pad = jnp.pad(x, ((0, n_pad - n), (0, 0)))
  eye_h = jnp.eye(d_hid, dtype=jnp.float32)
  eye_o = jnp.eye(d_out, dtype=jnp.float32)

  s1t, s1, dinv = pl.pallas_call(
      _dense1_body,
      out_shape=[jax.ShapeDtypeStruct((d_hid, n_pad), jnp.float32),
                 jax.ShapeDtypeStruct((n_pad, d_hid), jnp.float32),
                 jax.ShapeDtypeStruct((n_pad, 1), jnp.float32)],
  )(degp, x_pad, W1, eye_h)

  zeros_h = jnp.zeros((d_hid * n_pad,), jnp.float32)
  aggp1 = _make_colsplit_kernel(n_chunks, n_pad, d_hid)(
      s1t.reshape(-1), src_r, dst_r, zeros_h).reshape(NC, d_hid, n_pad)

  s2t, s2 = pl.pallas_call(
      _mid_body,
      out_shape=[jax.ShapeDtypeStruct((d_out, n_pad), jnp.float32),
                 jax.ShapeDtypeStruct((n_pad, d_out), jnp.float32)],
  )(aggp1, s1, dinv, b1.reshape(1, d_hid), W2, eye_h, eye_o)

  zeros_o = jnp.zeros((d_out * n_pad,), jnp.float32)
  aggp2 = _make_colsplit_kernel(n_chunks, n_pad, d_out)(
      s2t.reshape(-1), src_r, dst_r, zeros_o).reshape(NC, d_out, n_pad)

  z = pl.pallas_call(
      _final_body,
      out_shape=jax.ShapeDtypeStruct((n, d_out), jnp.float32),
  )(aggp2, s2, dinv, b2.reshape(1, d_out), eye_o)

  return z


# speedup vs baseline: 1.0984x; 1.0049x over previous
import functools

import jax
import jax.numpy as jnp
from jax import lax
from jax.experimental import pallas as pl
from jax.experimental.pallas import tpu as pltpu
from jax.experimental.pallas import tpu_sc as plsc

NC = 2
NS = 16
CH = 8192
DEGW = 16


def _sc_mesh():
  return plsc.VectorSubcoreMesh(core_axis_name="c", subcore_axis_name="s")


def _make_deg_kernel(K, n_pad):
  rpt = n_pad // NS

  @functools.partial(
      pl.kernel,
      out_type=jax.ShapeDtypeStruct((NC, n_pad, DEGW), jnp.float32),
      mesh=_sc_mesh(),
      scratch_types=[
          pltpu.VMEM((K, 128), jnp.int32),
          pltpu.VMEM((128, DEGW), jnp.float32),
          pltpu.VMEM_SHARED((n_pad, DEGW), jnp.float32),
      ],
      compiler_params=pltpu.CompilerParams(use_tc_tiling_on_sc=False),
  )
  def deg_kernel(dst_hbm, zeros_hbm, ones_hbm, degp_hbm, dst_v, ones_v, acc_sh):
    c = lax.axis_index("c")
    s = lax.axis_index("s")
    wid = s * NC + c
    pltpu.sync_copy(zeros_hbm.at[pl.ds(rpt * s, rpt)], acc_sh.at[pl.ds(rpt * s, rpt)])
    pltpu.sync_copy(ones_hbm, ones_v)
    pltpu.sync_copy(dst_hbm.at[wid], dst_v)
    plsc.subcore_barrier()

    def step(j, carry):
      pltpu.sync_copy(ones_v, acc_sh.at[dst_v.at[j]], add=True)
      return carry

    lax.fori_loop(0, K, step, 0)
    plsc.subcore_barrier()
    pltpu.sync_copy(acc_sh.at[pl.ds(rpt * s, rpt)],
                    degp_hbm.at[c, pl.ds(rpt * s, rpt)])

  return deg_kernel


def _make_colsplit_kernel(n_chunks, n_pad, d):
  assert d % NS == 0
  cpt = d // NS

  sl = cpt * n_pad

  @functools.partial(
      pl.kernel,
      out_type=jax.ShapeDtypeStruct((NC, d * n_pad), jnp.float32),
      mesh=_sc_mesh(),
      scratch_types=[
          pltpu.VMEM((sl,), jnp.float32),
          pltpu.VMEM((sl,), jnp.float32),
          pltpu.VMEM((2, 2, CH), jnp.int32),
          pltpu.SemaphoreType.DMA((2,)),
      ],
      compiler_params=pltpu.CompilerParams(use_tc_tiling_on_sc=False,
                                           needs_layout_passes=False),
  )
  def colsplit_kernel(st_hbm, idx_hbm, zeros_hbm, aggp_hbm,
                      tab_v, acc_v, idx_v, isem):
    c = lax.axis_index("c")
    t = lax.axis_index("s")
    pltpu.async_copy(idx_hbm.at[c, 0], idx_v.at[0], isem.at[0])
    pltpu.sync_copy(st_hbm.at[pl.ds(t * sl, sl)], tab_v)
    pltpu.sync_copy(zeros_hbm.at[pl.ds(t * sl, sl)], acc_v)

    def chunk_step(j, carry):
      p = lax.rem(j, 2)
      q = lax.rem(j + 1, 2)
      pltpu.make_async_copy(idx_hbm.at[c, j], idx_v.at[p], isem.at[p]).wait()

      @pl.when(j + 1 < n_chunks)
      def _():
        pltpu.async_copy(idx_hbm.at[c, j + 1], idx_v.at[q], isem.at[q])

      @plsc.parallel_loop(0, CH // 16, unroll=8)
      def _body(i):
        sv = idx_v[p, 0, pl.ds(i * 16, 16)]
        dv = idx_v[p, 1, pl.ds(i * 16, 16)]
        for col in range(cpt):
          if col == 0:
            si, di = sv, dv
          else:
            off = jnp.full((16,), col * n_pad, jnp.int32)
            si, di = sv + off, dv + off
          vals = plsc.load_gather(tab_v, [si])
          plsc.addupdate_scatter(acc_v, [di], vals)

      return carry

    lax.fori_loop(0, n_chunks, chunk_step, 0)
    pltpu.sync_copy(acc_v, aggp_hbm.at[c, pl.ds(t * sl, sl)])

  return colsplit_kernel


def _dense1_body(degp_ref, x_ref, w1_ref, eye_ref, s1t_ref, s1_ref, dinv_ref):
  deg = degp_ref[0, :, 0:1] + degp_ref[1, :, 0:1] + 1.0
  dinv = lax.rsqrt(deg)
  dinv_ref[...] = dinv
  h1 = jnp.dot(x_ref[...], w1_ref[...], preferred_element_type=jnp.float32)
  s1 = h1 * dinv
  s1_ref[...] = s1
  s1t_ref[...] = lax.dot_general(eye_ref[...], s1, (((1,), (1,)), ((), ())),
                                 preferred_element_type=jnp.float32,
                                 precision=lax.Precision.HIGHEST)


def _mid_body(aggp_ref, s1_ref, dinv_ref, b1_ref, w2_ref, eye_h_ref,
              eye_o_ref, s2t_ref, s2_ref):
  dinv = dinv_ref[...]
  agg_t = aggp_ref[0] + aggp_ref[1]
  agg = lax.dot_general(agg_t, eye_h_ref[...], (((0,), (0,)), ((), ())),
                        preferred_element_type=jnp.float32,
                        precision=lax.Precision.HIGHEST)
  out1 = jnp.maximum(dinv * (agg + s1_ref[...]) + b1_ref[...], 0.0)
  s2 = jnp.dot(out1, w2_ref[...], preferred_element_type=jnp.float32) * dinv
  s2_ref[...] = s2
  s2t_ref[...] = lax.dot_general(eye_o_ref[...], s2, (((1,), (1,)), ((), ())),
                                 preferred_element_type=jnp.float32,
                                 precision=lax.Precision.HIGHEST)


def _final_body(aggp_ref, s2_ref, dinv_ref, b2_ref, eye_ref, z_ref):
  n = z_ref.shape[0]
  agg_t = aggp_ref[0] + aggp_ref[1]
  agg = lax.dot_general(agg_t, eye_ref[...], (((0,), (0,)), ((), ())),
                        preferred_element_type=jnp.float32,
                        precision=lax.Precision.HIGHEST)
  zfull = dinv_ref[...] * (agg + s2_ref[...]) + b2_ref[...]
  z_ref[...] = zfull[:n, :]


def kernel(x, edge_index, W1, b1, W2, b2):
  n, d_in = x.shape
  d_hid = W1.shape[1]
  d_out = W2.shape[1]
  e = edge_index.shape[1]

  n_chunks = -(-e // (NC * CH))
  e_pad = NC * n_chunks * CH
  n_pad = -(-(n + 1) // 128) * 128

  pad = e_pad - e
  src = jnp.concatenate([edge_index[0], jnp.zeros((pad,), jnp.int32)])
  dst = jnp.concatenate([edge_index[1], jnp.full((pad,), n, jnp.int32)])
  idx_r = jnp.stack([src.reshape(NC, n_chunks, CH),
                     dst.reshape(NC, n_chunks, CH)], axis=2)

  K = -(-e // (NC * NS * 128))
  e_pad_d = NC * NS * K * 128
  dst_d = jnp.concatenate([edge_index[1], jnp.full((e_pad_d - e,), n, jnp.int32)])
  dst_dr = dst_d.reshape(NC * NS, K, 128)

  zeros_deg = jnp.zeros((n_pad, DEGW), jnp.float32)
  ones_col = jnp.ones((128, DEGW), jnp.float32)
  degp = _make_deg_kernel(K, n_pad)(dst_dr, zeros_deg, ones_col)

  x_pad = jnp.pad(x, ((0, n_pad - n), (0, 0)))
  eye_h = jnp.eye(d_hid, dtype=jnp.float32)
  eye_o = jnp.eye(d_out, dtype=jnp.float32)

  s1t, s1, dinv = pl.pallas_call(
      _dense1_body,
      out_shape=[jax.ShapeDtypeStruct((d_hid, n_pad), jnp.float32),
                 jax.ShapeDtypeStruct((n_pad, d_hid), jnp.float32),
                 jax.ShapeDtypeStruct((n_pad, 1), jnp.float32)],
  )(degp, x_pad, W1, eye_h)

  zeros_h = jnp.zeros((d_hid * n_pad,), jnp.float32)
  aggp1 = _make_colsplit_kernel(n_chunks, n_pad, d_hid)(
      s1t.reshape(-1), idx_r, zeros_h).reshape(NC, d_hid, n_pad)

  s2t, s2 = pl.pallas_call(
      _mid_body,
      out_shape=[jax.ShapeDtypeStruct((d_out, n_pad), jnp.float32),
                 jax.ShapeDtypeStruct((n_pad, d_out), jnp.float32)],
  )(aggp1, s1, dinv, b1.reshape(1, d_hid), W2, eye_h, eye_o)

  zeros_o = jnp.zeros((d_out * n_pad,), jnp.float32)
  aggp2 = _make_colsplit_kernel(n_chunks, n_pad, d_out)(
      s2t.reshape(-1), idx_r, zeros_o).reshape(NC, d_out, n_pad)

  z = pl.pallas_call(
      _final_body,
      out_shape=jax.ShapeDtypeStruct((n, d_out), jnp.float32),
  )(aggp2, s2, dinv, b2.reshape(1, d_out), eye_o)

  return z
